# Initial kernel scaffold; baseline (speedup 1.0000x reference)
import jax
import jax.numpy as jnp
from jax.experimental import pallas as pl


def kernel(x, edge_index, edge_weight, W_e1, b_e1, W_e2, b_e2, W_e3, b_e3, W_d1, b_d1, W_d2, b_d2, W_d3, b_d3):
    raise NotImplementedError("write your pallas kernel here")


# speedup vs baseline: 13.3763x; 13.3763x over previous
import functools

import jax
import jax.numpy as jnp
from jax import lax
from jax.experimental import pallas as pl
from jax.experimental.pallas import tpu as pltpu
from jax.experimental.pallas import tpu_sc as plsc

NN = 10000
NP = 10240
EE = 160000
IN = 3
HID = 512
NC = 2
NS = 16
LL = 16

ROWS_T = NP // NS
EPT32 = EE // (NC * NS)
EPT16 = EE // NS
NB512 = 25
BB512 = EPT16 // NB512
WB = 64

_MESH = plsc.VectorSubcoreMesh(core_axis_name="c", subcore_axis_name="s")
_SC_PARAMS = pltpu.CompilerParams(needs_layout_passes=False)
_MESH1 = plsc.VectorSubcoreMesh(core_axis_name="c", subcore_axis_name="s",
                                num_cores=1)

f32 = jnp.float32
i32 = jnp.int32


def _lrelu(v):
    return jnp.maximum(v, v * 0.01)


def _rsqrt_sc(d):
    bits = lax.bitcast_convert_type(d, i32)
    y = lax.bitcast_convert_type(jnp.int32(0x5F3759DF) - (bits >> 1), f32)
    for _ in range(3):
        y = y * (1.5 - 0.5 * d * y * y)
    return y


@functools.partial(
    pl.kernel,
    out_type=jax.ShapeDtypeStruct((NC, NP), f32),
    mesh=_MESH,
    scratch_types=[
        pltpu.VMEM((EPT32,), i32),
        pltpu.VMEM((EPT32,), f32),
        pltpu.VMEM((ROWS_T,), f32),
        pltpu.VMEM_SHARED((NP,), f32),
    ],
    compiler_params=_SC_PARAMS,
    name="gcn_pre_deg",
)
def _pre_deg(dst_hbm, ew_hbm, deg2_hbm, dst_v, ew_v, zb_v, acc_s):
    core = lax.axis_index("c")
    sub = lax.axis_index("s")
    tid = core * NS + sub
    r0 = pl.multiple_of(sub * ROWS_T, 8)
    for j in range(ROWS_T // LL):
        zb_v[pl.ds(j * LL, LL)] = jnp.zeros((LL,), f32)
    pltpu.sync_copy(zb_v, acc_s.at[pl.ds(r0, ROWS_T)])
    plsc.subcore_barrier()
    pltpu.sync_copy(dst_hbm.at[tid], dst_v)
    pltpu.sync_copy(ew_hbm.at[tid], ew_v)
    pltpu.sync_copy(ew_v, acc_s.at[dst_v], add=True)
    plsc.subcore_barrier()
    pltpu.sync_copy(acc_s.at[pl.ds(r0, ROWS_T)],
                    deg2_hbm.at[core, pl.ds(r0, ROWS_T)])


@functools.partial(
    pl.kernel,
    out_type=(jax.ShapeDtypeStruct((NC * NS, EPT32), f32),
              jax.ShapeDtypeStruct((NP,), f32)),
    mesh=_MESH,
    scratch_types=[
        pltpu.VMEM((NP,), f32),
        pltpu.VMEM((NP,), f32),
        pltpu.VMEM((EPT32,), i32),
        pltpu.VMEM((EPT32,), i32),
        pltpu.VMEM((EPT32,), f32),
        pltpu.VMEM((EPT32,), f32),
        pltpu.VMEM((NP // (NC * NS),), f32),
    ],
    compiler_params=_SC_PARAMS,
    name="gcn_pre_coef",
)
def _pre_coef(deg2_hbm, src_hbm, dst_hbm, ew_hbm, coef_hbm, selfc_hbm,
              da_v, db_v, src_v, dst_v, ew_v, cf_v, sq_v):
    core = lax.axis_index("c")
    sub = lax.axis_index("s")
    tid = core * NS + sub
    pltpu.sync_copy(deg2_hbm.at[0], da_v)
    pltpu.sync_copy(deg2_hbm.at[1], db_v)

    def dinv_body(j, _):
        sl = pl.ds(pl.multiple_of(j * LL, LL), LL)
        d = da_v[sl] + db_v[sl] + 1.0
        da_v[sl] = _rsqrt_sc(d)
        return 0
    lax.fori_loop(0, NP // LL, dinv_body, 0)

    m = NP // (NC * NS)
    s0 = pl.multiple_of(tid * m, 8)

    def sq_body(j, _):
        v = da_v[pl.ds(s0 + j * LL, LL)]
        sq_v[pl.ds(pl.multiple_of(j * LL, LL), LL)] = v * v
        return 0
    lax.fori_loop(0, m // LL, sq_body, 0)
    pltpu.sync_copy(sq_v, selfc_hbm.at[pl.ds(s0, m)])

    pltpu.sync_copy(src_hbm.at[tid], src_v)
    pltpu.sync_copy(dst_hbm.at[tid], dst_v)
    pltpu.sync_copy(ew_hbm.at[tid], ew_v)

    def coef_chunk(off):
        sv = src_v[pl.ds(off, LL)]
        dv = dst_v[pl.ds(off, LL)]
        wv = ew_v[pl.ds(off, LL)]
        ds_ = plsc.load_gather(da_v, [sv])
        dd_ = plsc.load_gather(da_v, [dv])
        cf_v[pl.ds(off, LL)] = ds_ * wv * dd_

    def coef_body(j, _):
        coef_chunk(pl.multiple_of(j * LL, LL))
        return 0
    lax.fori_loop(0, EPT32 // LL, coef_body, 0)
    if EPT32 % LL:
        coef_chunk(EPT32 - LL)
    pltpu.sync_copy(cf_v, coef_hbm.at[tid])


def _make_p3(out_w8, act):
    ow = 4 if out_w8 else 2

    @functools.partial(
        pl.kernel,
        out_type=jax.ShapeDtypeStruct((NC * NP * ow,), f32),
        mesh=_MESH,
        scratch_types=[
            pltpu.VMEM((NP * 4,), f32),
            pltpu.VMEM((EPT16,), i32),
            pltpu.VMEM((EPT16,), i32),
            pltpu.VMEM((EPT16,), f32),
            pltpu.VMEM((EPT16 * 2,), f32),
            pltpu.VMEM((EPT16 * 2,), i32),
            pltpu.VMEM((ROWS_T * 2,), f32),
            pltpu.VMEM((ROWS_T * 2,), f32),
            pltpu.VMEM((ROWS_T * ow,), f32),
            pltpu.VMEM((LL,), f32),
            pltpu.VMEM_SHARED((NP * 2,), f32),
        ],
        compiler_params=_SC_PARAMS,
        name="gcn_p3" + ("_w8" if out_w8 else "_w4") + ("_act" if act else ""),
    )
    def body(t_hbm, src_hbm, dst_hbm, coef_hbm, selfc2_hbm, bp_hbm, out_hbm,
             t_v, src_v, dst_v, coef_v, msg_v, didx_v, rbuf_v, sc_v, obuf_v,
             bp_v, acc_s):
        core = lax.axis_index("c")
        sub = lax.axis_index("s")
        cA = core * 2
        r0 = pl.multiple_of(sub * ROWS_T, 8)
        a0 = pl.multiple_of(r0 * 2, 8)
        lane = lax.iota(i32, LL)
        zeros = jnp.zeros((LL,), f32)

        def z_body(j, _):
            rbuf_v[pl.ds(pl.multiple_of(j * LL, LL), LL)] = zeros
            return 0
        lax.fori_loop(0, ROWS_T * 2 // LL, z_body, 0)
        pltpu.sync_copy(rbuf_v, acc_s.at[pl.ds(a0, ROWS_T * 2)])

        pltpu.sync_copy(t_hbm, t_v)
        pltpu.sync_copy(src_hbm.at[sub], src_v)
        pltpu.sync_copy(dst_hbm.at[sub], dst_v)
        pltpu.sync_copy(coef_hbm.at[sub], coef_v)
        pltpu.sync_copy(selfc2_hbm.at[pl.ds(a0, ROWS_T * 2)], sc_v)
        pltpu.sync_copy(bp_hbm.at[core], bp_v)
        if out_w8:
            def oz_body(j, _):
                obuf_v[pl.ds(pl.multiple_of(j * LL, LL), LL)] = zeros
                return 0
            lax.fori_loop(0, ROWS_T * ow // LL, oz_body, 0)
        plsc.subcore_barrier()

        def e_body(j, _):
            off = pl.multiple_of(j * LL, LL)
            off2 = pl.multiple_of(off * 2, LL)
            sv = src_v[pl.ds(off, LL)]
            cv = coef_v[pl.ds(off, LL)]
            dv = dst_v[pl.ds(off, LL)]
            base = sv * 4 + cA
            gA = plsc.load_gather(t_v, [base])
            gB = plsc.load_gather(t_v, [base + 1])
            msg_v[pl.ds(off2, LL)] = gA * cv
            msg_v[pl.ds(off2 + LL, LL)] = gB * cv
            d2 = dv * 2
            didx_v[pl.ds(off2, LL)] = d2
            didx_v[pl.ds(off2 + LL, LL)] = d2 + 1
            return 0
        lax.fori_loop(0, EPT16 // LL, e_body, 0)
        pltpu.sync_copy(msg_v, acc_s.at[didx_v], add=True)
        plsc.subcore_barrier()

        pltpu.sync_copy(acc_s.at[pl.ds(a0, ROWS_T * 2)], rbuf_v)
        bb = bp_v[pl.ds(0, LL)]
        row_l = lane >> 1
        col_l = lane & 1

        def w_body(j, _):
            sl = pl.ds(pl.multiple_of(j * LL, LL), LL)
            rv = rbuf_v[sl]
            scv = sc_v[sl]
            rl = j * 8 + row_l
            tv = plsc.load_gather(t_v, [(r0 + rl) * 4 + cA + col_l])
            val = rv + scv * tv + bb
            if act:
                val = _lrelu(val)
            if out_w8:
                plsc.store_scatter(obuf_v, [rl * 4 + col_l], val)
            else:
                obuf_v[sl] = val
            return 0
        lax.fori_loop(0, ROWS_T // 8, w_body, 0)
        o0 = pl.multiple_of((core * NP + r0) * ow, 8)
        pltpu.sync_copy(obuf_v, out_hbm.at[pl.ds(o0, ROWS_T * ow)])

    return body


_p3_plain_w8 = _make_p3(True, False)
_p3_act_w4 = _make_p3(False, True)
_p3_noact_w4 = _make_p3(False, False)


NB2 = 32
BKN = NP // NB2
EPAD = 512
QPAD = 3072
NQ4 = NP // 4
BST = 128
NSTEP = NS * EPAD // BST

@functools.partial(
    pl.kernel,
    out_type=(jax.ShapeDtypeStruct((NB2, NS, EPAD), i32),
              jax.ShapeDtypeStruct((NB2, NS, EPAD), i32),
              jax.ShapeDtypeStruct((NB2, NS, EPAD), f32),
              jax.ShapeDtypeStruct((NC, NS, LL), i32)),
    mesh=_MESH,
    scratch_types=[
        pltpu.VMEM((EPT16,), i32),
        pltpu.VMEM((EPT16,), i32),
        pltpu.VMEM((EPT16,), f32),
        pltpu.VMEM((QPAD + LL,), i32),
        pltpu.VMEM((QPAD + LL,), i32),
        pltpu.VMEM((QPAD + LL,), f32),
        pltpu.VMEM((EPAD + LL,), i32),
        pltpu.VMEM((EPAD + LL,), i32),
        pltpu.VMEM((EPAD + LL,), f32),
        pltpu.VMEM((LL,), i32),
    ],
    compiler_params=_SC_PARAMS,
    name="gcn_part",
)
def _pre_part(src_hbm, dst_hbm, coef_hbm, srcp_hbm, dstp_hbm, cfp_hbm,
              cnts_hbm, src_v, dst_v, coef_v, sq_v, dq_v, cq_v, so_v, do_v,
              co_v, cnt_v):
    core = lax.axis_index("c")
    sub = lax.axis_index("s")
    lane = lax.iota(i32, LL)
    pltpu.sync_copy(src_hbm.at[sub], src_v)
    pltpu.sync_copy(dst_hbm.at[sub], dst_v)
    pltpu.sync_copy(coef_hbm.at[sub], coef_v)

    for qq in range(2):
        q = core * 2 + qq
        qbase = q * NQ4

        def f1_body(j, _):
            sl = pl.ds(pl.multiple_of(j * LL, LL), LL)
            sq_v[sl] = (j * LL + lane) & 8191
            dq_v[sl] = lane * 0
            cq_v[sl] = jnp.zeros((LL,), f32)
            return 0
        lax.fori_loop(0, (QPAD + LL) // LL, f1_body, 0)

        def c1_body(j, pos):
            off = pl.multiple_of(j * LL, LL)
            sv = src_v[pl.ds(off, LL)]
            dv = dst_v[pl.ds(off, LL)]
            cv = coef_v[pl.ds(off, LL)]
            mask = (dv >= qbase) & (dv < qbase + NQ4)
            w = pl.ds(pos, LL)
            plsc.store_compressed(sq_v.at[w], sv, mask=mask)
            plsc.store_compressed(dq_v.at[w], dv - qbase, mask=mask)
            plsc.store_compressed(cq_v.at[w], cv, mask=mask)
            return pos + plsc.all_reduce_population_count(mask)[0]
        pos1 = lax.fori_loop(0, EPT16 // LL, c1_body, jnp.int32(0))
        n1 = (pos1 + LL - 1) // LL

        for sb in range(8):
            b2 = sb * BKN

            def f2_body(j, _):
                sl = pl.ds(pl.multiple_of(j * LL, LL), LL)
                so_v[sl] = (j * LL + lane) & 8191
                do_v[sl] = (j * LL + lane) & 255
                co_v[sl] = jnp.zeros((LL,), f32)
                return 0
            lax.fori_loop(0, (EPAD + LL) // LL, f2_body, 0)

            def c2_body(j, pos):
                off = j * LL
                sv = sq_v[pl.ds(off, LL)]
                dv = dq_v[pl.ds(off, LL)]
                cv = cq_v[pl.ds(off, LL)]
                mask = (dv >= b2) & (dv < b2 + BKN)
                w = pl.ds(pos, LL)
                plsc.store_compressed(so_v.at[w], sv, mask=mask)
                plsc.store_compressed(do_v.at[w], dv - b2, mask=mask)
                plsc.store_compressed(co_v.at[w], cv, mask=mask)
                return pos + plsc.all_reduce_population_count(mask)[0]
            pos2 = lax.fori_loop(0, n1, c2_body, jnp.int32(0))
            lb = qq * 8 + sb
            plsc.store_scatter(cnt_v, [lane * 0 + lb],
                               lane * 0 + pos2, mask=lane == lb)

            bk = q * 8 + sb
            pltpu.sync_copy(so_v.at[pl.ds(0, EPAD)], srcp_hbm.at[bk, sub])
            pltpu.sync_copy(do_v.at[pl.ds(0, EPAD)], dstp_hbm.at[bk, sub])
            pltpu.sync_copy(co_v.at[pl.ds(0, EPAD)], cfp_hbm.at[bk, sub])
    pltpu.sync_copy(cnt_v, cnts_hbm.at[core, sub])


@functools.partial(
    pl.kernel,
    out_type=jax.ShapeDtypeStruct((4, NP, 128), f32),
    mesh=_MESH,
    scratch_types=[
        pltpu.VMEM((NS * EPAD,), i32),
        pltpu.VMEM((NS * EPAD,), i32),
        pltpu.VMEM((NS * EPAD,), f32),
        pltpu.VMEM((BST,), i32),
        pltpu.VMEM((BST,), i32),
        pltpu.VMEM((BST,), i32),
        pltpu.VMEM((BST, 128), f32),
        pltpu.VMEM((BST, 128), f32),
        pltpu.VMEM((BST, 128), f32),
        pltpu.VMEM((BKN, 128), f32),
        pltpu.VMEM((BKN,), i32),
        pltpu.VMEM((BKN,), f32),
        pltpu.VMEM((128,), f32),
        pltpu.VMEM((LL,), i32),
        pltpu.VMEM((NSTEP + LL,), i32),
        pltpu.SemaphoreType.DMA,
        pltpu.SemaphoreType.DMA,
        pltpu.SemaphoreType.DMA,
    ],
    compiler_params=_SC_PARAMS,
    name="gcn_p512",
)
def _p512(t_hbm, srcp_hbm, dstp_hbm, cfp_hbm, cnts_hbm, selfc_hbm, bias_hbm,
          out_hbm, src_v, dst_v, coef_v, idxa_v, idxb_v, idxc_v,
          msga_v, msgb_v, msgc_v, acc_v, oidx_v, sc_v, bias_v,
          cnts_v, trips_v, sema, semb, semc):
    core = lax.axis_index("c")
    sub = lax.axis_index("s")
    lane = lax.iota(i32, LL)

    def rmw(msg_v, base, step):
        ntr = trips_v[pl.ds(step, LL)][0]

        def r_body(j16, _):
            off = base + j16 * LL
            cv = coef_v[pl.ds(off, LL)]
            dlv = dst_v[pl.ds(off, LL)]
            for jj in range(LL):
                c = cv[jj]
                dl = dlv[jj]
                row = j16 * LL + jj
                prods = [c * msg_v[row, pl.ds(k * LL, LL)] for k in range(8)]
                for k in range(8):
                    plsc.addupdate(acc_v.at[dl, pl.ds(k * LL, LL)], prods[k])
            return 0
        lax.fori_loop(0, ntr, r_body, 0)

    def cc_body(cc, _):
        chunk = core * 2 + cc
        pltpu.sync_copy(bias_hbm.at[chunk], bias_v)

        def h_body(h, _):
            bucket = sub * 2 + h
            pltpu.sync_copy(srcp_hbm.at[bucket], src_v)
            pltpu.sync_copy(dstp_hbm.at[bucket], dst_v)
            pltpu.sync_copy(cfp_hbm.at[bucket], coef_v)
            pltpu.sync_copy(selfc_hbm.at[pl.ds(bucket * BKN, BKN)], sc_v)
            pltpu.sync_copy(cnts_hbm.at[bucket], cnts_v)

            def t_body(t16, _):
                steps = t16 * LL + lane
                sli = steps >> 2
                o = (steps & 3) * BST
                cg = plsc.load_gather(cnts_v, [sli])
                trip = (cg - o + LL - 1) >> 4
                trip = jnp.maximum(trip, 0)
                trip = jnp.minimum(trip, BST // LL)
                trips_v[pl.ds(pl.multiple_of(t16 * LL, LL), LL)] = trip
                return 0
            lax.fori_loop(0, NSTEP // LL, t_body, 0)
            trips_v[pl.ds(NSTEP, LL)] = lane * 0

            def za_body(j, _):
                for k in range(8):
                    acc_v[j, pl.ds(k * LL, LL)] = jnp.zeros((LL,), f32)
                return 0
            lax.fori_loop(0, BKN, za_body, 0)

            tc_ref = t_hbm.at[chunk]
            bufs = ((idxa_v, msga_v, sema), (idxb_v, msgb_v, semb),
                    (idxc_v, msgc_v, semc))

            def gather(step, mb, sb_):
                return pltpu.async_copy(
                    tc_ref.at[src_v.at[pl.ds(step * BST, BST)]], mb, sb_)

            def gwait(step, mb, sb_):
                pltpu.make_async_copy(
                    tc_ref.at[src_v.at[pl.ds(step * BST, BST)]],
                    mb, sb_).wait()

            for i, (ib, mb, sb_) in enumerate(bufs):
                gather(i, mb, sb_)

            def s_body(s, _):
                for i, (ib, mb, sb_) in enumerate(bufs):
                    step = 3 * s + i
                    gwait(step, mb, sb_)
                    rmw(mb, step * BST, step)

                    @pl.when(step + 3 < NSTEP)
                    def _():
                        gather(step + 3, mb, sb_)
                return 0
            lax.fori_loop(0, NSTEP // 3, s_body, 0)
            gwait(NSTEP - 1, msga_v, sema)
            rmw(msga_v, (NSTEP - 1) * BST, NSTEP - 1)

            def o_body(j, _):
                off = pl.multiple_of(j * LL, LL)
                oidx_v[pl.ds(off, LL)] = bucket * BKN + j * LL + lane
                return 0
            lax.fori_loop(0, BKN // LL, o_body, 0)
            for i, (ib, mb, sb_) in enumerate(bufs):
                n = 64 if i == 2 else BST
                pltpu.async_copy(
                    tc_ref.at[oidx_v.at[pl.ds(i * BST, n)]],
                    mb.at[pl.ds(0, n)], sb_)
            for i, (ib, mb, sb_) in enumerate(bufs):
                n = 64 if i == 2 else BST
                pltpu.make_async_copy(
                    tc_ref.at[oidx_v.at[pl.ds(i * BST, n)]],
                    mb.at[pl.ds(0, n)], sb_).wait()

                def f_body(j16, _):
                    scv16 = sc_v[pl.ds(i * BST + j16 * LL, LL)]
                    for jj in range(LL):
                        scv = scv16[jj]
                        row = j16 * LL + jj
                        ar = i * BST + row
                        for k in range(8):
                            sl = pl.ds(k * LL, LL)
                            val = (acc_v[ar, sl] + scv * mb[row, sl]
                                   + bias_v[sl])
                            acc_v[ar, sl] = _lrelu(val)
                    return 0
                lax.fori_loop(0, n // LL, f_body, 0)
            pltpu.sync_copy(acc_v, out_hbm.at[chunk].at[oidx_v])
            return 0
        lax.fori_loop(0, 2, h_body, 0)
        return 0
    lax.fori_loop(0, 2, cc_body, 0)


BR = 1024


def _dmm(p8, W8, b1, W2):
    def body(p_ref, w8_ref, b_ref, w2_ref, o_ref):
        h = jnp.dot(p_ref[...], w8_ref[...], preferred_element_type=f32)
        h = h + b_ref[...]
        h = jnp.maximum(h, h * 0.01)
        o_ref[0] = jnp.dot(h, w2_ref[...], preferred_element_type=f32)

    return pl.pallas_call(
        body,
        grid=(NP // BR, 4),
        in_specs=[
            pl.BlockSpec((BR, 8), lambda i, j: (i, 0)),
            pl.BlockSpec((8, HID), lambda i, j: (0, 0)),
            pl.BlockSpec((1, HID), lambda i, j: (0, 0)),
            pl.BlockSpec((HID, 128), lambda i, j: (0, j)),
        ],
        out_specs=pl.BlockSpec((1, BR, 128), lambda i, j: (j, i, 0)),
        out_shape=jax.ShapeDtypeStruct((4, NP, 128), f32),
        name="gcn_dmm",
    )(p8, W8, b1, W2)


def _mm128(h4, Wp4):
    def body(h_ref, w_ref, o_ref):
        o = jnp.dot(h_ref[0], w_ref[0], preferred_element_type=f32)
        for c in range(1, 4):
            o = o + jnp.dot(h_ref[c], w_ref[c], preferred_element_type=f32)
        o_ref[...] = o

    return pl.pallas_call(
        body,
        grid=(NP // BR,),
        in_specs=[
            pl.BlockSpec((4, BR, 128), lambda i: (0, i, 0)),
            pl.BlockSpec((4, 128, 128), lambda i: (0, 0, 0)),
        ],
        out_specs=pl.BlockSpec((BR, 128), lambda i: (i, 0)),
        out_shape=jax.ShapeDtypeStruct((NP, 128), f32),
        name="gcn_mm128",
    )(h4, Wp4)


def _bias_pat(b4):
    return jnp.stack([jnp.tile(b4[0:2], 8), jnp.tile(b4[2:4], 8)])


def _w8(W):
    z = jnp.zeros((HID,), f32)
    return jnp.stack([W[0], W[1], z, z, W[2], z, z, z])


def kernel(x, edge_index, edge_weight,
           W_e1, b_e1, W_e2, b_e2, W_e3, b_e3,
           W_d1, b_d1, W_d2, b_d2, W_d3, b_d3):
    src = edge_index[0]
    dst = edge_index[1]
    src32 = src.reshape(NC * NS, EPT32)
    dst32 = dst.reshape(NC * NS, EPT32)
    ew32 = edge_weight.reshape(NC * NS, EPT32)

    deg2 = _pre_deg(dst32, ew32)
    coef32, selfc = _pre_coef(deg2, src32, dst32, ew32)
    coef = coef32.reshape(EE)

    src16 = src.reshape(NS, EPT16)
    dst16 = dst.reshape(NS, EPT16)
    coef16 = coef.reshape(NS, EPT16)
    srcp, dstp, cfp, cnts = _pre_part(src16, dst16, coef16)
    srcp = srcp.reshape(NB2, NS * EPAD)
    dstp = dstp.reshape(NB2, NS * EPAD)
    cfp = cfp.reshape(NB2, NS * EPAD)
    cnts_t = cnts.transpose(0, 2, 1).reshape(NB2, NS)
    selfc2 = jnp.broadcast_to(selfc[:, None], (NP, 2)).reshape(NP * 2)

    zpat = _bias_pat(jnp.zeros((4,), f32))
    b_e3p = _bias_pat(jnp.concatenate([b_e3, jnp.zeros((1,), f32)]))
    b_d3p = _bias_pat(jnp.concatenate([b_d3, jnp.zeros((1,), f32)]))

    x4 = jnp.pad(x, ((0, NP - NN), (0, 1)))

    def interleave(o_flat, ow):
        return o_flat.reshape(NC, NP, ow).transpose(1, 0, 2).reshape(NP, 2 * ow)

    p0 = _p3_plain_w8(x4.reshape(NP * 4), src16, dst16, coef16, selfc2, zpat)
    t2 = _dmm(interleave(p0, 4), _w8(W_e1), b_e1.reshape(1, HID), W_e2)
    h2 = _p512(t2, srcp, dstp, cfp, cnts_t, selfc, b_e2.reshape(4, 128))
    t3 = _mm128(h2, jnp.pad(W_e3, ((0, 0), (0, 125))).reshape(4, 128, 128))
    h3 = _p3_act_w4(t3[:, :4].reshape(NP * 4), src16, dst16, coef16, selfc2,
                    b_e3p)
    p4 = _p3_plain_w8(interleave(h3, 2).reshape(NP * 4), src16, dst16, coef16,
                      selfc2, zpat)
    t5 = _dmm(interleave(p4, 4), _w8(W_d1), b_d1.reshape(1, HID), W_d2)
    h5 = _p512(t5, srcp, dstp, cfp, cnts_t, selfc, b_d2.reshape(4, 128))
    t6 = _mm128(h5, jnp.pad(W_d3, ((0, 0), (0, 125))).reshape(4, 128, 128))
    out4 = _p3_noact_w4(t6[:, :4].reshape(NP * 4), src16, dst16, coef16,
                        selfc2, b_d3p)
    return interleave(out4, 2)[:NN, :IN]
